# baseline (speedup 1.0000x reference)
import numpy as np
import jax
import jax.numpy as jnp
from jax.experimental import pallas as pl
from jax.experimental.pallas import tpu as pltpu

_MEM = 16
_NIMG = 32

_E1 = (np.arange(84)[None, None, :] ==
       4 * np.arange(20)[None, :, None] + np.arange(8)[:, None, None]
       ).astype(np.float32)
_E2 = (np.arange(20)[None, None, :] ==
       2 * np.arange(9)[None, :, None] + np.arange(4)[:, None, None]
       ).astype(np.float32)
_E3 = (np.arange(9)[None, None, :] ==
       np.arange(7)[None, :, None] + np.arange(3)[:, None, None]
       ).astype(np.float32)

_OFF_W1 = 0
_OFF_W2 = 2 * 1344
_OFF_W3 = _OFF_W2 + 4 * 640
_OFF_WD = _OFF_W3 + 3 * 576
_ROWS_W = _OFF_WD + 7 * 448


def _sel(shape, expr):
    return expr.astype(jnp.bfloat16)


def _tower_kernel(x_ref, w_ref, b_ref, o_ref):
    n = _NIMG
    r1 = n * 21
    r2 = n * 10
    f32 = jnp.float32

    bf16 = jnp.bfloat16
    xb = x_ref[...].astype(bf16)
    dot = lambda a, bm: jax.lax.dot_general(
        a, bm, (((1,), (0,)), ((), ())), preferred_element_type=f32)

    p0 = dot(xb, w_ref[_OFF_W1:_OFF_W1 + 1344, :])
    p1 = dot(xb, w_ref[_OFF_W1 + 1344:_OFF_W1 + 2688, :])
    ra = jax.lax.broadcasted_iota(jnp.int32, (r1, r1), 0)
    ca = jax.lax.broadcasted_iota(jnp.int32, (r1, r1), 1)
    s1 = _sel((r1, r1), ca == ra + 1)
    p1b = p1.astype(bf16)
    y1 = jnp.maximum(p0 + dot(s1, p1b) + b_ref[0:1, :], 0.0).astype(bf16)

    rb = jax.lax.broadcasted_iota(jnp.int32, (r2, r1), 0)
    cb = jax.lax.broadcasted_iota(jnp.int32, (r2, r1), 1)
    src0 = 21 * (rb // 10) + 2 * (rb % 10)
    acc2 = b_ref[1:2, :576]
    for t in range(4):
        st = _sel((r2, r1), cb == src0 + t)
        wt = w_ref[_OFF_W2 + t * 640:_OFF_W2 + (t + 1) * 640, :576]
        acc2 = acc2 + dot(dot(st, y1).astype(bf16), wt)
    y2 = jnp.maximum(acc2, 0.0).astype(bf16)

    rc = jax.lax.broadcasted_iota(jnp.int32, (r2, r2), 0)
    cc = jax.lax.broadcasted_iota(jnp.int32, (r2, r2), 1)
    acc3 = b_ref[2:3, :448]
    for t in range(3):
        st = _sel((r2, r2), cc == rc + t)
        wt = w_ref[_OFF_W3 + t * 576:_OFF_W3 + (t + 1) * 576, :448]
        acc3 = acc3 + dot(dot(st, y2).astype(bf16), wt)
    y3 = jnp.maximum(acc3, 0.0).astype(bf16)

    rd = jax.lax.broadcasted_iota(jnp.int32, (n, r2), 0)
    cd = jax.lax.broadcasted_iota(jnp.int32, (n, r2), 1)
    accd = b_ref[3:4, :512]
    for t in range(7):
        st = _sel((n, r2), cd == 10 * rd + t)
        wt = w_ref[_OFF_WD + t * 448:_OFF_WD + (t + 1) * 448, :512]
        accd = accd + dot(dot(st, y3).astype(bf16), wt)
    o_ref[...] = jnp.maximum(accd, 0.0)


def _pack_weights(c1_w, c2_w, c3_w, dense_w):
    w1 = c1_w.reshape(2, 4, 8, 4, 32)
    w1b = [jnp.einsum('rkio,kjc->rcijo', w1[d], _E1).reshape(1344, 640)
           for d in range(2)]
    w2b = [jnp.pad(
        jnp.einsum('cio,cqj->jiqo', c2_w[t], _E2).reshape(640, 576),
        ((0, 0), (0, 64))) for t in range(4)]
    w3b = [jnp.pad(
        jnp.einsum('wio,wqj->jiqo', c3_w[t], _E3).reshape(576, 448),
        ((0, 0), (0, 192))) for t in range(3)]
    wdb = [jnp.pad(dense_w.reshape(7, 448, 512)[t], ((0, 0), (0, 128)))
           for t in range(7)]
    return jnp.concatenate(w1b + w2b + w3b + wdb,
                           axis=0).astype(jnp.bfloat16)


def _gru_kernel(x_ref, d_ref, h0_ref, s0_ref, wk_ref, rk_ref, bias_ref,
                seq_ref, hf_ref, sf_ref, h_scr, s_scr):
    t = pl.program_id(1)

    @pl.when(t == 0)
    def _init():
        h_scr[...] = h0_ref[...]
        s_scr[...] = s0_ref[...]

    x = x_ref[0]
    h = h_scr[...]
    wk = wk_ref[...]
    rk = rk_ref[...]
    bias = bias_ref[...]

    xz = jnp.dot(x, wk[0:512], preferred_element_type=jnp.float32)
    xr = jnp.dot(x, wk[512:1024], preferred_element_type=jnp.float32)
    xh = jnp.dot(x, wk[1024:1536], preferred_element_type=jnp.float32)
    rz = jnp.dot(h, rk[0:64], preferred_element_type=jnp.float32)
    rr = jnp.dot(h, rk[64:128], preferred_element_type=jnp.float32)
    rh = jnp.dot(h, rk[128:192], preferred_element_type=jnp.float32)

    z = jax.nn.sigmoid(xz + bias[0:1] + rz + bias[3:4])
    r = jax.nn.sigmoid(xr + bias[1:2] + rr + bias[4:5])
    hh = jnp.tanh(xh + bias[2:3] + r * (rh + bias[5:6]))
    h_new = z * h + (1.0 - z) * hh

    seq_ref[0] = h_new

    step = s_scr[...] + 1
    d = d_ref[0]
    reset = jnp.logical_or(d == 1, step % _MEM == 0)
    h_next = jnp.where(reset, 0.0, h_new)
    s_next = jnp.where(reset, 0, step)
    h_scr[...] = h_next
    s_scr[...] = s_next
    hf_ref[...] = h_next
    sf_ref[...] = s_next


def kernel(inputs, dones, state0, step0, c1_w, c1_b, c2_w, c2_b, c3_w, c3_b,
           dense_w, dense_b, gru_k, gru_rk, gru_b):
    b, t = inputs.shape[:2]
    bt = b * t

    xb = inputs.reshape(bt * 21, 1344)
    wpack = _pack_weights(c1_w, c2_w, c3_w, dense_w)
    bias = jnp.stack([
        jnp.tile(c1_b, 20),
        jnp.pad(jnp.tile(c2_b, 9), (0, 64)),
        jnp.pad(jnp.tile(c3_b, 7), (0, 192)),
        jnp.pad(dense_b, (0, 128)),
    ], axis=0)

    feats = pl.pallas_call(
        _tower_kernel,
        grid=(bt // _NIMG,),
        in_specs=[
            pl.BlockSpec((_NIMG * 21, 1344), lambda i: (i, 0)),
            pl.BlockSpec((_ROWS_W, 640), lambda i: (0, 0)),
            pl.BlockSpec((4, 640), lambda i: (0, 0)),
        ],
        out_specs=pl.BlockSpec((_NIMG, 512), lambda i: (i, 0)),
        out_shape=jax.ShapeDtypeStruct((bt, 512), jnp.float32),
        compiler_params=pltpu.CompilerParams(
            dimension_semantics=("parallel",)),
    )(xb, wpack, bias)

    units = state0.shape[1]
    feats_tm = feats.reshape(b, t, 512).transpose(1, 0, 2)
    dones_tm = dones.transpose(1, 0)[:, :, None]
    wk_all = gru_k.transpose(1, 0).reshape(3, units, 512)
    wk_all = wk_all.transpose(0, 2, 1).reshape(3 * 512, units)
    rk_all = gru_rk.transpose(1, 0).reshape(3, units, units)
    rk_all = rk_all.transpose(0, 2, 1).reshape(3 * units, units)
    gbias = jnp.concatenate(
        [gru_b[0].reshape(3, units), gru_b[1].reshape(3, units)], axis=0)
    bh = b // 2

    seq, state_f, step_f = pl.pallas_call(
        _gru_kernel,
        grid=(2, t),
        in_specs=[
            pl.BlockSpec((1, bh, 512), lambda g, s: (s, g, 0)),
            pl.BlockSpec((1, bh, 1), lambda g, s: (s, g, 0)),
            pl.BlockSpec((bh, units), lambda g, s: (g, 0)),
            pl.BlockSpec((bh, 1), lambda g, s: (g, 0)),
            pl.BlockSpec((3 * 512, units), lambda g, s: (0, 0)),
            pl.BlockSpec((3 * units, units), lambda g, s: (0, 0)),
            pl.BlockSpec((6, units), lambda g, s: (0, 0)),
        ],
        out_specs=[
            pl.BlockSpec((1, bh, units), lambda g, s: (s, g, 0)),
            pl.BlockSpec((bh, units), lambda g, s: (g, 0)),
            pl.BlockSpec((bh, 1), lambda g, s: (g, 0)),
        ],
        out_shape=[
            jax.ShapeDtypeStruct((t, b, units), jnp.float32),
            jax.ShapeDtypeStruct((b, units), jnp.float32),
            jax.ShapeDtypeStruct((b, 1), jnp.int32),
        ],
        scratch_shapes=[
            pltpu.VMEM((bh, units), jnp.float32),
            pltpu.VMEM((bh, 1), jnp.int32),
        ],
        compiler_params=pltpu.CompilerParams(
            dimension_semantics=("parallel", "arbitrary")),
    )(feats_tm, dones_tm, state0, step0[:, None], wk_all, rk_all, gbias)

    y = seq.transpose(1, 0, 2)
    out = jnp.concatenate([feats.reshape(b, t, 512), y], axis=2)
    return (out, state_f, step_f.reshape(b))


# speedup vs baseline: 4.9855x; 4.9855x over previous
import numpy as np
import jax
import jax.numpy as jnp
from jax.experimental import pallas as pl
from jax.experimental.pallas import tpu as pltpu

_MEM = 16
_NIMG = 32

_E1 = (4 * np.arange(21)[None, None, :, None] + np.arange(4)[None, None, None, :] ==
       4 * np.arange(20)[None, :, None, None] + np.arange(8)[:, None, None, None]
       ).astype(np.float32)
_E2 = (np.arange(20)[None, None, :] ==
       2 * np.arange(9)[None, :, None] + np.arange(4)[:, None, None]
       ).astype(np.float32)
_E3 = (np.arange(9)[None, None, :] ==
       np.arange(7)[None, :, None] + np.arange(3)[:, None, None]
       ).astype(np.float32)

_OFF_W1 = 0
_OFF_W2 = 2 * 1344
_OFF_W3 = _OFF_W2 + 4 * 640
_OFF_WD = _OFF_W3 + 3 * 576
_ROWS_W = _OFF_WD + 7 * 448


def _sel(shape, expr):
    return expr.astype(jnp.bfloat16)


def _tower_kernel(x_ref, w_ref, b_ref, o_ref):
    n = _NIMG
    r1 = n * 21
    r2 = n * 10
    f32 = jnp.float32

    bf16 = jnp.bfloat16
    xb = x_ref[...].astype(bf16)
    dot = lambda a, bm: jax.lax.dot_general(
        a, bm, (((1,), (0,)), ((), ())), preferred_element_type=f32)

    p0 = dot(xb, w_ref[_OFF_W1:_OFF_W1 + 1344, :])
    p1 = dot(xb, w_ref[_OFF_W1 + 1344:_OFF_W1 + 2688, :])
    ra = jax.lax.broadcasted_iota(jnp.int32, (r1, r1), 0)
    ca = jax.lax.broadcasted_iota(jnp.int32, (r1, r1), 1)
    s1 = _sel((r1, r1), ca == ra + 1)
    p1b = p1.astype(bf16)
    y1 = jnp.maximum(p0 + dot(s1, p1b) + b_ref[0:1, :], 0.0).astype(bf16)

    rb = jax.lax.broadcasted_iota(jnp.int32, (r2, r1), 0)
    cb = jax.lax.broadcasted_iota(jnp.int32, (r2, r1), 1)
    src0 = 21 * (rb // 10) + 2 * (rb % 10)
    acc2 = b_ref[1:2, :576]
    for t in range(4):
        st = _sel((r2, r1), cb == src0 + t)
        wt = w_ref[_OFF_W2 + t * 640:_OFF_W2 + (t + 1) * 640, :576]
        acc2 = acc2 + dot(dot(st, y1).astype(bf16), wt)
    y2 = jnp.maximum(acc2, 0.0).astype(bf16)

    rc = jax.lax.broadcasted_iota(jnp.int32, (r2, r2), 0)
    cc = jax.lax.broadcasted_iota(jnp.int32, (r2, r2), 1)
    acc3 = b_ref[2:3, :448]
    for t in range(3):
        st = _sel((r2, r2), cc == rc + t)
        wt = w_ref[_OFF_W3 + t * 576:_OFF_W3 + (t + 1) * 576, :448]
        acc3 = acc3 + dot(dot(st, y2).astype(bf16), wt)
    y3 = jnp.maximum(acc3, 0.0).astype(bf16)

    rd = jax.lax.broadcasted_iota(jnp.int32, (n, r2), 0)
    cd = jax.lax.broadcasted_iota(jnp.int32, (n, r2), 1)
    accd = b_ref[3:4, :512]
    for t in range(7):
        st = _sel((n, r2), cd == 10 * rd + t)
        wt = w_ref[_OFF_WD + t * 448:_OFF_WD + (t + 1) * 448, :512]
        accd = accd + dot(dot(st, y3).astype(bf16), wt)
    o_ref[...] = jnp.maximum(accd, 0.0)


def _pack_weights(c1_w, c2_w, c3_w, dense_w):
    w1 = c1_w.reshape(2, 4, 8, 4, 32)
    w1b = [jnp.einsum('rkio,kjbc->brcijo', w1[d], _E1).reshape(1344, 640)
           for d in range(2)]
    w2b = [jnp.pad(
        jnp.einsum('cio,cqj->jiqo', c2_w[t], _E2).reshape(640, 576),
        ((0, 0), (0, 64))) for t in range(4)]
    w3b = [jnp.pad(
        jnp.einsum('wio,wqj->jiqo', c3_w[t], _E3).reshape(576, 448),
        ((0, 0), (0, 192))) for t in range(3)]
    wdb = [jnp.pad(dense_w.reshape(7, 448, 512)[t], ((0, 0), (0, 128)))
           for t in range(7)]
    return jnp.concatenate(w1b + w2b + w3b + wdb,
                           axis=0).astype(jnp.bfloat16)


def _gru_kernel(x_ref, d_ref, h0_ref, s0_ref, wk_ref, rk_ref, bias_ref,
                seq_ref, hf_ref, sf_ref, h_scr, s_scr):
    t = pl.program_id(1)

    @pl.when(t == 0)
    def _init():
        h_scr[...] = h0_ref[...]
        s_scr[...] = s0_ref[...]

    x = x_ref[0]
    h = h_scr[...]
    wk = wk_ref[...]
    rk = rk_ref[...]
    bias = bias_ref[...]

    xz = jnp.dot(x, wk[0:512], preferred_element_type=jnp.float32)
    xr = jnp.dot(x, wk[512:1024], preferred_element_type=jnp.float32)
    xh = jnp.dot(x, wk[1024:1536], preferred_element_type=jnp.float32)
    rz = jnp.dot(h, rk[0:64], preferred_element_type=jnp.float32)
    rr = jnp.dot(h, rk[64:128], preferred_element_type=jnp.float32)
    rh = jnp.dot(h, rk[128:192], preferred_element_type=jnp.float32)

    z = jax.nn.sigmoid(xz + bias[0:1] + rz + bias[3:4])
    r = jax.nn.sigmoid(xr + bias[1:2] + rr + bias[4:5])
    hh = jnp.tanh(xh + bias[2:3] + r * (rh + bias[5:6]))
    h_new = z * h + (1.0 - z) * hh

    seq_ref[0] = h_new

    step = s_scr[...] + 1
    d = d_ref[0]
    reset = jnp.logical_or(d == 1, step % _MEM == 0)
    h_next = jnp.where(reset, 0.0, h_new)
    s_next = jnp.where(reset, 0, step)
    h_scr[...] = h_next
    s_scr[...] = s_next
    hf_ref[...] = h_next
    sf_ref[...] = s_next


def kernel(inputs, dones, state0, step0, c1_w, c1_b, c2_w, c2_b, c3_w, c3_b,
           dense_w, dense_b, gru_k, gru_rk, gru_b):
    b, t = inputs.shape[:2]
    bt = b * t

    xb = inputs.reshape(bt, 21, 4, 21, 4, 4).transpose(0, 1, 3, 2, 4, 5)
    xb = xb.reshape(bt * 21, 1344)
    wpack = _pack_weights(c1_w, c2_w, c3_w, dense_w)
    bias = jnp.stack([
        jnp.tile(c1_b, 20),
        jnp.pad(jnp.tile(c2_b, 9), (0, 64)),
        jnp.pad(jnp.tile(c3_b, 7), (0, 192)),
        jnp.pad(dense_b, (0, 128)),
    ], axis=0)

    feats = pl.pallas_call(
        _tower_kernel,
        grid=(bt // _NIMG,),
        in_specs=[
            pl.BlockSpec((_NIMG * 21, 1344), lambda i: (i, 0)),
            pl.BlockSpec((_ROWS_W, 640), lambda i: (0, 0)),
            pl.BlockSpec((4, 640), lambda i: (0, 0)),
        ],
        out_specs=pl.BlockSpec((_NIMG, 512), lambda i: (i, 0)),
        out_shape=jax.ShapeDtypeStruct((bt, 512), jnp.float32),
        compiler_params=pltpu.CompilerParams(
            dimension_semantics=("parallel",)),
    )(xb, wpack, bias)

    units = state0.shape[1]
    feats_tm = feats.reshape(b, t, 512).transpose(1, 0, 2)
    dones_tm = dones.transpose(1, 0)[:, :, None]
    wk_all = gru_k.transpose(1, 0).reshape(3, units, 512)
    wk_all = wk_all.transpose(0, 2, 1).reshape(3 * 512, units)
    rk_all = gru_rk.transpose(1, 0).reshape(3, units, units)
    rk_all = rk_all.transpose(0, 2, 1).reshape(3 * units, units)
    gbias = jnp.concatenate(
        [gru_b[0].reshape(3, units), gru_b[1].reshape(3, units)], axis=0)
    bh = b // 2

    seq, state_f, step_f = pl.pallas_call(
        _gru_kernel,
        grid=(2, t),
        in_specs=[
            pl.BlockSpec((1, bh, 512), lambda g, s: (s, g, 0)),
            pl.BlockSpec((1, bh, 1), lambda g, s: (s, g, 0)),
            pl.BlockSpec((bh, units), lambda g, s: (g, 0)),
            pl.BlockSpec((bh, 1), lambda g, s: (g, 0)),
            pl.BlockSpec((3 * 512, units), lambda g, s: (0, 0)),
            pl.BlockSpec((3 * units, units), lambda g, s: (0, 0)),
            pl.BlockSpec((6, units), lambda g, s: (0, 0)),
        ],
        out_specs=[
            pl.BlockSpec((1, bh, units), lambda g, s: (s, g, 0)),
            pl.BlockSpec((bh, units), lambda g, s: (g, 0)),
            pl.BlockSpec((bh, 1), lambda g, s: (g, 0)),
        ],
        out_shape=[
            jax.ShapeDtypeStruct((t, b, units), jnp.float32),
            jax.ShapeDtypeStruct((b, units), jnp.float32),
            jax.ShapeDtypeStruct((b, 1), jnp.int32),
        ],
        scratch_shapes=[
            pltpu.VMEM((bh, units), jnp.float32),
            pltpu.VMEM((bh, 1), jnp.int32),
        ],
        compiler_params=pltpu.CompilerParams(
            dimension_semantics=("parallel", "arbitrary")),
    )(feats_tm, dones_tm, state0, step0[:, None], wk_all, rk_all, gbias)

    y = seq.transpose(1, 0, 2)
    out = jnp.concatenate([feats.reshape(b, t, 512), y], axis=2)
    return (out, state_f, step_f.reshape(b))
